# baseline (speedup 1.0000x reference)
import jax
import jax.numpy as jnp
from jax.experimental import pallas as pl
from jax.experimental.pallas import tpu as pltpu

_B, _C, _H, _W = 16, 192, 64, 64
_HW = _H * _W


def _gain_scale_body(n_ref, gm_ref, x_ref, o_ref):
    b = pl.program_id(0)
    nb = n_ref[b]
    nf = jnp.floor(nb)
    l = nb - nf
    ni = nf.astype(jnp.int32)
    g1 = jnp.abs(gm_ref[pl.ds(ni, 1), :])
    g2 = jnp.abs(gm_ref[pl.ds(ni + 1, 1), :])
    gain = g1 ** (1.0 - l) * g2 ** l
    o_ref[0] = x_ref[0] * gain.reshape(_C, 1)


def kernel(x, n, gain_matrix):
    xf = x.reshape(_B, _C, _HW)
    out = pl.pallas_call(
        _gain_scale_body,
        grid=(_B,),
        in_specs=[
            pl.BlockSpec(memory_space=pltpu.SMEM),
            pl.BlockSpec((6, _C), lambda b: (0, 0)),
            pl.BlockSpec((1, _C, _HW), lambda b: (b, 0, 0)),
        ],
        out_specs=pl.BlockSpec((1, _C, _HW), lambda b: (b, 0, 0)),
        out_shape=jax.ShapeDtypeStruct((_B, _C, _HW), jnp.float32),
    )(n, gain_matrix, xf)
    return out.reshape(_B, _C, _H, _W)
